# baseline (speedup 1.0000x reference)
import jax
import jax.numpy as jnp
from jax.experimental import pallas as pl
from jax.experimental.pallas import tpu as pltpu


def _body(v_ref, x_ref, y_ref, o_ref, pos_ref):
    W = x_ref.shape[0]
    H = y_ref.shape[0]
    D = x_ref.shape[1]

    @pl.when(pl.program_id(0) == 0)
    def _build_pos():
        pos = x_ref[...][:, None, :] + y_ref[...][None, :, :]
        pos_ref[...] = pos.reshape(W * H, D)

    o_ref[...] = v_ref[pl.ds((pl.program_id(0) % 2) * 4, 4)] + pos_ref[...][None]


def kernel(visn_feats, x_table, y_table):
    B, C, W, H = visn_feats.shape
    S = W * H
    D = x_table.shape[1]
    v = jnp.transpose(visn_feats, (0, 2, 3, 1)).reshape(B, S, C)
    return pl.pallas_call(
        _body,
        grid=(B // 4,),
        in_specs=[
            pl.BlockSpec((8, S, C), lambda b: (b // 2, 0, 0)),
            pl.BlockSpec((W, D), lambda b: (0, 0)),
            pl.BlockSpec((H, D), lambda b: (0, 0)),
        ],
        out_specs=pl.BlockSpec((4, S, C), lambda b: (b, 0, 0)),
        out_shape=jax.ShapeDtypeStruct((B, S, C), visn_feats.dtype),
        scratch_shapes=[pltpu.VMEM((S, D), visn_feats.dtype)],
    )(v, x_table, y_table)


# speedup vs baseline: 1.2376x; 1.2376x over previous
import jax
import jax.numpy as jnp
from jax.experimental import pallas as pl
from jax.experimental.pallas import tpu as pltpu


def _body(v_ref, x_ref, y_ref, o_ref, pos_ref):
    W = x_ref.shape[0]
    H = y_ref.shape[0]
    D = x_ref.shape[1]

    @pl.when(pl.program_id(0) == 0)
    def _build_pos():
        pos = x_ref[...][:, None, :] + y_ref[...][None, :, :]
        pos_ref[...] = pos.reshape(W * H, D)

    o_ref[...] = v_ref[...] + pos_ref[...][None]


def kernel(visn_feats, x_table, y_table):
    B, C, W, H = visn_feats.shape
    S = W * H
    D = x_table.shape[1]
    v = jnp.transpose(visn_feats, (0, 2, 3, 1)).reshape(B, S, C)
    return pl.pallas_call(
        _body,
        grid=(B // 8,),
        in_specs=[
            pl.BlockSpec((8, S, C), lambda b: (b, 0, 0)),
            pl.BlockSpec((W, D), lambda b: (0, 0)),
            pl.BlockSpec((H, D), lambda b: (0, 0)),
        ],
        out_specs=pl.BlockSpec((8, S, C), lambda b: (b, 0, 0)),
        out_shape=jax.ShapeDtypeStruct((B, S, C), visn_feats.dtype),
        scratch_shapes=[pltpu.VMEM((S, D), visn_feats.dtype)],
    )(v, x_table, y_table)
